# baseline (speedup 1.0000x reference)
import math

import numpy as np
import jax
import jax.numpy as jnp
from jax import lax
from jax.experimental import pallas as pl
from jax.experimental.pallas import tpu as pltpu
from jax.experimental.pallas import tpu_sc as plsc

_VOCAB = 100000
_D = 768
_B = 1024
_S = 200
_NC = 2
_NS = 16
_NW = _NC * _NS
_BPW = _B // _NW
_NV = _D // 16
_CSL = 4
_NSL = _S // _CSL
_NTP = _NSL // 2


def _coeff_tables():
    half = _D // 2
    freq = np.exp(-np.arange(half, dtype=np.float64) / half * math.log(10000.0))
    ang = np.arange(_S, dtype=np.float64)[:, None] * freq[None, :]
    c = np.cos(ang)
    s = np.sin(ang)
    a = np.repeat(1.0 + c, 2, axis=1).astype(np.float32)
    bp = np.stack([-s, s], axis=-1).reshape(_S, _D).astype(np.float32)
    return np.stack([a, bp], axis=1)


_C_TAB = _coeff_tables()


def _body(emb_hbm, idx4_hbm, c_hbm, out_hbm,
          rb0, rb1, rb2, rb3, cb0, cb1, is0, is1, oi0, oi1, oi2, oi3,
          gs0, gs1, gs2, gs3, ss0, ss1, ss2, ss3, cs0, cs1, es0, es1):
    wid = lax.axis_index("s") * _NC + lax.axis_index("c")
    ob_base = wid * _BPW * _S
    perm = (lax.iota(jnp.int32, 16) ^ 1)[:, None]
    dnums = lax.GatherDimensionNumbers(
        offset_dims=(), collapsed_slice_dims=(0,), start_index_map=(0,))
    jlo = lax.iota(jnp.int32, 16) * _S
    jhi = jlo + 16 * _S
    rb = (rb0, rb1, rb2, rb3)
    cb = (cb0, cb1)
    isl = (is0, is1)
    oi = (oi0, oi1, oi2, oi3)
    gs = (gs0, gs1, gs2, gs3)
    ss = (ss0, ss1, ss2, ss3)
    cs = (cs0, cs1)
    es = (es0, es1)

    def issue_gather(u, q, p):
        pltpu.async_copy(emb_hbm.at[isl[q].at[u]], rb[p], gs[p])

    def wait_gather(u, q, p):
        pltpu.make_async_copy(emb_hbm.at[isl[q].at[u]], rb[p], gs[p]).wait()

    def issue_slab(t, q):
        pltpu.async_copy(c_hbm.at[pl.ds(t * _CSL, _CSL)], cb[q], cs[q])
        pltpu.async_copy(idx4_hbm.at[t, wid], isl[q], es[q])

    def drain_cslab(q):
        pltpu.make_async_copy(c_hbm.at[pl.ds(0, _CSL)], cb[q], cs[q]).wait()

    def drain_islab(q):
        pltpu.make_async_copy(idx4_hbm.at[0, 0], isl[q], es[q]).wait()

    def issue_scatter(p):
        pltpu.async_copy(rb[p], out_hbm.at[oi[p]], ss[p])

    def drain_scatter(p):
        pltpu.make_async_copy(rb[p], out_hbm.at[oi[p]], ss[p]).wait()

    def compute(p, q, u):
        rbuf, cbuf = rb[p], cb[q]

        def kfn(k, carry):
            ksl = pl.ds(pl.multiple_of(k * 16, 16), 16)
            a = cbuf[u, 0, ksl]
            bb = cbuf[u, 1, ksl]
            for j in range(_BPW):
                x = rbuf[j, ksl]
                rbuf[j, ksl] = x + a * 0.0 + bb * 0.0
            return carry

        lax.fori_loop(0, _NV, kfn, 0)

    pltpu.sync_copy(idx4_hbm.at[0, wid], is0)
    pltpu.async_copy(c_hbm.at[pl.ds(0, _CSL)], cb0, cs0)
    issue_gather(0, 0, 0)
    issue_gather(1, 0, 1)

    _ahead = [(0, 2), (0, 3), (1, 0), (1, 1), (1, 2), (1, 3), (0, 0), (0, 1)]

    def pair(tp, carry):
        for v in range(8):
            tpar, u = divmod(v, _CSL)
            p = v % 4
            pf = (v + 2) % 4
            if v == 0:
                drain_cslab(0)
                issue_slab(2 * tp + 1, 1)
            elif v == 4:
                drain_cslab(1)

                @pl.when(tp <= _NTP - 2)
                def _():
                    issue_slab(2 * tp + 2, 0)
            wait_gather(u, tpar, p)
            aq, au = _ahead[v]
            if v == 2:
                drain_islab(1)
            if v < 6:
                if v < 2:
                    @pl.when(tp >= 1)
                    def _():
                        drain_scatter(pf)
                else:
                    drain_scatter(pf)
                issue_gather(au, aq, pf)
            else:
                @pl.when(tp <= _NTP - 2)
                def _():
                    if v == 6:
                        drain_islab(0)
                    drain_scatter(pf)
                    issue_gather(au, aq, pf)
            compute(p, tpar, u)
            base = ob_base + 8 * tp + v
            oi[p][pl.ds(0, 16)] = jlo + base
            oi[p][pl.ds(16, 16)] = jhi + base
            issue_scatter(p)
        return carry

    lax.fori_loop(0, _NTP, pair, 0)
    for p in range(4):
        drain_scatter(p)


def kernel(sequence, emb):
    idx4 = sequence.T.reshape(_NSL, _CSL, _NW, _BPW).transpose(0, 2, 1, 3)
    c_tab = jnp.asarray(_C_TAB)
    mesh = plsc.VectorSubcoreMesh(core_axis_name="c", subcore_axis_name="s",
                                  num_cores=_NC, num_subcores=_NS)
    out = pl.kernel(
        _body,
        out_type=jax.ShapeDtypeStruct((_B * _S, _D), jnp.float32),
        mesh=mesh,
        scratch_types=[
            pltpu.VMEM((_BPW, _D), jnp.float32),
            pltpu.VMEM((_BPW, _D), jnp.float32),
            pltpu.VMEM((_BPW, _D), jnp.float32),
            pltpu.VMEM((_BPW, _D), jnp.float32),
            pltpu.VMEM((_CSL, 2, _D), jnp.float32),
            pltpu.VMEM((_CSL, 2, _D), jnp.float32),
            pltpu.VMEM((_CSL, _BPW), jnp.int32),
            pltpu.VMEM((_CSL, _BPW), jnp.int32),
            pltpu.VMEM((_BPW,), jnp.int32),
            pltpu.VMEM((_BPW,), jnp.int32),
            pltpu.VMEM((_BPW,), jnp.int32),
            pltpu.VMEM((_BPW,), jnp.int32),
            pltpu.SemaphoreType.DMA,
            pltpu.SemaphoreType.DMA,
            pltpu.SemaphoreType.DMA,
            pltpu.SemaphoreType.DMA,
            pltpu.SemaphoreType.DMA,
            pltpu.SemaphoreType.DMA,
            pltpu.SemaphoreType.DMA,
            pltpu.SemaphoreType.DMA,
            pltpu.SemaphoreType.DMA,
            pltpu.SemaphoreType.DMA,
            pltpu.SemaphoreType.DMA,
            pltpu.SemaphoreType.DMA,
        ],
    )(emb, idx4, c_tab)
    return out.reshape(_B, _S, _D)


# speedup vs baseline: 1.0005x; 1.0005x over previous
import math

import numpy as np
import jax
import jax.numpy as jnp
from jax import lax
from jax.experimental import pallas as pl
from jax.experimental.pallas import tpu as pltpu
from jax.experimental.pallas import tpu_sc as plsc

_VOCAB = 100000
_D = 768
_B = 1024
_S = 200
_NC = 2
_NS = 16
_NW = _NC * _NS
_BPW = _B // _NW
_NV = _D // 16
_CSL = 4
_NSL = _S // _CSL
_NTP = _NSL // 2


def _coeff_tables():
    half = _D // 2
    freq = np.exp(-np.arange(half, dtype=np.float64) / half * math.log(10000.0))
    ang = np.arange(_S, dtype=np.float64)[:, None] * freq[None, :]
    c = np.cos(ang)
    s = np.sin(ang)
    a = np.repeat(1.0 + c, 2, axis=1).astype(np.float32)
    bp = np.stack([-s, s], axis=-1).reshape(_S, _D).astype(np.float32)
    return np.stack([a, bp], axis=1)


_C_TAB = _coeff_tables()


def _body(emb_hbm, idx4_hbm, c_hbm, out_hbm,
          rb0, rb1, rb2, rb3, cb0, cb1, is0, is1, oi0, oi1, oi2, oi3,
          gs0, gs1, gs2, gs3, ss0, ss1, ss2, ss3, cs0, cs1, es0, es1):
    wid = lax.axis_index("s") * _NC + lax.axis_index("c")
    ob_base = wid * _BPW * _S
    perm = (lax.iota(jnp.int32, 16) ^ 1)[:, None]
    dnums = lax.GatherDimensionNumbers(
        offset_dims=(), collapsed_slice_dims=(0,), start_index_map=(0,))
    jlo = lax.iota(jnp.int32, 16) * _S
    jhi = jlo + 16 * _S
    rb = (rb0, rb1, rb2, rb3)
    cb = (cb0, cb1)
    isl = (is0, is1)
    oi = (oi0, oi1, oi2, oi3)
    gs = (gs0, gs1, gs2, gs3)
    ss = (ss0, ss1, ss2, ss3)
    cs = (cs0, cs1)
    es = (es0, es1)

    def issue_gather(u, q, p):
        pltpu.async_copy(emb_hbm.at[isl[q].at[u]], rb[p], gs[p])

    def wait_gather(u, q, p):
        pltpu.make_async_copy(emb_hbm.at[isl[q].at[u]], rb[p], gs[p]).wait()

    def issue_slab(t, q):
        pltpu.async_copy(c_hbm.at[pl.ds(t * _CSL, _CSL)], cb[q], cs[q])
        pltpu.async_copy(idx4_hbm.at[t, wid], isl[q], es[q])

    def drain_cslab(q):
        pltpu.make_async_copy(c_hbm.at[pl.ds(0, _CSL)], cb[q], cs[q]).wait()

    def drain_islab(q):
        pltpu.make_async_copy(idx4_hbm.at[0, 0], isl[q], es[q]).wait()

    def issue_scatter(p):
        pltpu.async_copy(rb[p], out_hbm.at[oi[p]], ss[p])

    def drain_scatter(p):
        pltpu.make_async_copy(rb[p], out_hbm.at[oi[p]], ss[p]).wait()

    def compute(p, q, u):
        rbuf, cbuf = rb[p], cb[q]

        def kfn(k, carry):
            ksl = pl.ds(pl.multiple_of(k * 16, 16), 16)
            a = cbuf[u, 0, ksl]
            bb = cbuf[u, 1, ksl]
            for j in range(_BPW):
                x = rbuf[j, ksl]
                xs = lax.gather(x, perm, dnums, slice_sizes=(1,),
                                unique_indices=True,
                                mode=lax.GatherScatterMode.PROMISE_IN_BOUNDS)
                rbuf[j, ksl] = x * a + xs * bb
            return carry

        lax.fori_loop(0, _NV, kfn, 0)

    pltpu.sync_copy(idx4_hbm.at[0, wid], is0)
    pltpu.async_copy(c_hbm.at[pl.ds(0, _CSL)], cb0, cs0)
    issue_gather(0, 0, 0)
    issue_gather(1, 0, 1)

    _ahead = [(0, 2), (0, 3), (1, 0), (1, 1), (1, 2), (1, 3), (0, 0), (0, 1)]

    def pair(tp, carry):
        for v in range(8):
            tpar, u = divmod(v, _CSL)
            p = v % 4
            pf = (v + 2) % 4
            if v == 0:
                drain_cslab(0)
                issue_slab(2 * tp + 1, 1)
            elif v == 4:
                drain_cslab(1)

                @pl.when(tp <= _NTP - 2)
                def _():
                    issue_slab(2 * tp + 2, 0)
            wait_gather(u, tpar, p)
            aq, au = _ahead[v]
            if v == 2:
                drain_islab(1)
            if v < 6:
                if v < 2:
                    @pl.when(tp >= 1)
                    def _():
                        drain_scatter(pf)
                else:
                    drain_scatter(pf)
                issue_gather(au, aq, pf)
            else:
                @pl.when(tp <= _NTP - 2)
                def _():
                    if v == 6:
                        drain_islab(0)
                    drain_scatter(pf)
                    issue_gather(au, aq, pf)
            compute(p, tpar, u)
            base = ob_base + 8 * tp + v
            oi[p][pl.ds(0, 16)] = jlo + base
            oi[p][pl.ds(16, 16)] = jhi + base
            issue_scatter(p)
        return carry

    lax.fori_loop(0, _NTP, pair, 0)
    for p in range(4):
        drain_scatter(p)


def kernel(sequence, emb):
    idx4 = sequence.T.reshape(_NSL, _CSL, _NW, _BPW).transpose(0, 2, 1, 3)
    c_tab = jnp.asarray(_C_TAB)
    mesh = plsc.VectorSubcoreMesh(core_axis_name="c", subcore_axis_name="s",
                                  num_cores=_NC, num_subcores=_NS)
    out = pl.kernel(
        _body,
        out_type=jax.ShapeDtypeStruct((_B * _S, _D), jnp.float32),
        mesh=mesh,
        scratch_types=[
            pltpu.VMEM((_BPW, _D), jnp.float32),
            pltpu.VMEM((_BPW, _D), jnp.float32),
            pltpu.VMEM((_BPW, _D), jnp.float32),
            pltpu.VMEM((_BPW, _D), jnp.float32),
            pltpu.VMEM((_CSL, 2, _D), jnp.float32),
            pltpu.VMEM((_CSL, 2, _D), jnp.float32),
            pltpu.VMEM((_CSL, _BPW), jnp.int32),
            pltpu.VMEM((_CSL, _BPW), jnp.int32),
            pltpu.VMEM((_BPW,), jnp.int32),
            pltpu.VMEM((_BPW,), jnp.int32),
            pltpu.VMEM((_BPW,), jnp.int32),
            pltpu.VMEM((_BPW,), jnp.int32),
            pltpu.SemaphoreType.DMA,
            pltpu.SemaphoreType.DMA,
            pltpu.SemaphoreType.DMA,
            pltpu.SemaphoreType.DMA,
            pltpu.SemaphoreType.DMA,
            pltpu.SemaphoreType.DMA,
            pltpu.SemaphoreType.DMA,
            pltpu.SemaphoreType.DMA,
            pltpu.SemaphoreType.DMA,
            pltpu.SemaphoreType.DMA,
            pltpu.SemaphoreType.DMA,
            pltpu.SemaphoreType.DMA,
        ],
    )(emb, idx4, c_tab)
    return out.reshape(_B, _S, _D)
